# Initial kernel scaffold; baseline (speedup 1.0000x reference)
import jax
import jax.numpy as jnp
from jax.experimental import pallas as pl


def kernel(x, intensity_):
    raise NotImplementedError("write your pallas kernel here")


# speedup vs baseline: 4.7521x; 4.7521x over previous
import functools

import jax
import jax.numpy as jnp
import numpy as np
from jax import lax
from jax.experimental import pallas as pl
from jax.experimental.pallas import tpu as pltpu
from jax.experimental.pallas import tpu_sc as plsc

_B = 128
_S = 8192
_NCHUNK = _S // 16
_NSUP = 32
_CPS = _NCHUNK // _NSUP
_ROWS_PER_W = 4
_NEG = float("-inf")


def _sel16_traced():
    keys = jax.random.split(jax.random.key(1), _B)
    perms = jax.vmap(lambda k: jax.random.permutation(k, 5))(keys)
    sel = perms[:, :2].astype(jnp.int32)
    return jnp.concatenate([sel, jnp.zeros((_B, 14), jnp.int32)], axis=1)


def _tree_max16(load):
    vs = [load(c) for c in range(16)]
    while len(vs) > 1:
        vs = [jnp.maximum(vs[i], vs[i + 1]) for i in range(0, len(vs), 2)]
    return vs[0]


def _sc_body(int_hbm, sel_hbm, pos_hbm, row_v, msup_v, sel_v, res_v):
    wid = lax.axis_index("s") * 2 + lax.axis_index("c")
    iota16 = jnp.arange(16, dtype=jnp.int32)

    for r in range(_ROWS_PER_W):
        row = wid * _ROWS_PER_W + r
        pltpu.sync_copy(int_hbm.at[row], row_v)
        pltpu.sync_copy(sel_hbm.at[row], sel_v)

        def super_body(s, acc):
            m = _tree_max16(lambda c: row_v[s * _CPS + c, :])
            msup_v[s, :] = m
            return jnp.maximum(acc, m)

        acc = lax.fori_loop(0, _NSUP, super_body,
                            jnp.full((16,), _NEG, jnp.float32))

        pos_list = []
        for k in range(5):
            gmax = jnp.max(acc)
            gv = jnp.full((16,), gmax)

            def fs_body(s, fs):
                hit = jnp.any(msup_v[s, :] == gv)
                return jnp.where((fs == 999) & hit, s, fs)

            fs = lax.fori_loop(0, _NSUP, fs_body, jnp.int32(999))

            def fc_body(c, carry):
                fc, fl = carry
                eqm = row_v[fs * _CPS + c, :] == gv
                hit = jnp.any(eqm)
                lane = jnp.min(jnp.where(eqm, iota16, 99))
                take = (fc == 999) & hit
                return (jnp.where(take, c, fc), jnp.where(take, lane, fl))

            fc, fl = lax.fori_loop(0, _CPS, fc_body,
                                   (jnp.int32(999), jnp.int32(0)))

            chunk = fs * _CPS + fc
            pos_list.append(chunk * 16 + fl)

            v = row_v[chunk, :]
            row_v[chunk, :] = jnp.where(iota16 == fl, _NEG, v)
            msup_v[fs, :] = _tree_max16(lambda c: row_v[fs * _CPS + c, :])
            acc = lax.fori_loop(
                0, _NSUP,
                lambda s, a: jnp.maximum(a, msup_v[s, :]),
                jnp.full((16,), _NEG, jnp.float32))

        sv = sel_v[...]
        res = jnp.zeros((16,), jnp.int32)
        for rank in range(5):
            res = jnp.where(sv == rank, pos_list[rank], res)
        res = jnp.where(iota16 < 2, res, 0)
        res_v[...] = res
        pltpu.sync_copy(res_v, pos_hbm.at[row])


_sc_top5 = functools.partial(
    pl.kernel,
    mesh=plsc.VectorSubcoreMesh(core_axis_name="c", subcore_axis_name="s"),
    compiler_params=pltpu.CompilerParams(needs_layout_passes=False),
    out_type=jax.ShapeDtypeStruct((_B, 16), jnp.int32),
    scratch_types=[
        pltpu.VMEM((_NCHUNK, 16), jnp.float32),
        pltpu.VMEM((_NSUP, 16), jnp.float32),
        pltpu.VMEM((16,), jnp.int32),
        pltpu.VMEM((16,), jnp.int32),
    ],
)(_sc_body)

_ROWBLK = 8


def _tc_body(pos_ref, x_ref, out_ref, tok_ref):
    x = x_ref[...]
    pos = pos_ref[...]
    p0 = pos[:, 0:1]
    p1 = pos[:, 1:2]
    col = lax.broadcasted_iota(jnp.int32, (_ROWBLK, _S), 1)
    m0 = col == p0
    m1 = col == p1
    out_ref[...] = jnp.where(m0 | m1, jnp.float32(1.0), x)
    t0 = jnp.sum(jnp.where(m0, x, 0.0), axis=1, keepdims=True).astype(jnp.int32)
    t1 = jnp.sum(jnp.where(m1, x, 0.0), axis=1, keepdims=True).astype(jnp.int32)
    col8 = lax.broadcasted_iota(jnp.int32, (_ROWBLK, 8), 1)
    tok_ref[...] = jnp.where(col8 == 0, t0, jnp.where(col8 == 1, t1, 0))


_tc_mask = pl.pallas_call(
    _tc_body,
    grid=(_B // _ROWBLK,),
    in_specs=[
        pl.BlockSpec((_ROWBLK, 16), lambda i: (i, 0)),
        pl.BlockSpec((_ROWBLK, _S), lambda i: (i, 0)),
    ],
    out_specs=[
        pl.BlockSpec((_ROWBLK, _S), lambda i: (i, 0)),
        pl.BlockSpec((_ROWBLK, 8), lambda i: (i, 0)),
    ],
    out_shape=[
        jax.ShapeDtypeStruct((_B, _S), jnp.float32),
        jax.ShapeDtypeStruct((_B, 8), jnp.int32),
    ],
)


def kernel(x, intensity_):
    intens = jnp.squeeze(intensity_, axis=1).reshape(_B, _NCHUNK, 16)
    sel16 = _sel16_traced()
    pos16 = _sc_top5(intens, sel16)
    mask_x, tok = _tc_mask(pos16, x)
    return (mask_x, tok, pos16[:, :8])


# speedup vs baseline: 6.6873x; 1.4072x over previous
import functools

import jax
import jax.numpy as jnp
import numpy as np
from jax import lax
from jax.experimental import pallas as pl
from jax.experimental.pallas import tpu as pltpu
from jax.experimental.pallas import tpu_sc as plsc

_B = 128
_S = 8192
_NCHUNK = _S // 16
_NSUP = 32
_CPS = _NCHUNK // _NSUP
_ROWS_PER_W = 4
_NEG = float("-inf")


def _sel16_traced():
    keys = jax.random.split(jax.random.key(1), _B)
    perms = jax.vmap(lambda k: jax.random.permutation(k, 5))(keys)
    sel = perms[:, :2].astype(jnp.int32)
    return jnp.concatenate([sel, jnp.zeros((_B, 14), jnp.int32)], axis=1)


def _tree_max16(load):
    vs = [load(c) for c in range(16)]
    while len(vs) > 1:
        vs = [jnp.maximum(vs[i], vs[i + 1]) for i in range(0, len(vs), 2)]
    return vs[0]


def _sc_body(int_hbm, sel_hbm, pos_hbm, row_v, msup_v, sel_v, res_v):
    wid = lax.axis_index("s") * 2 + lax.axis_index("c")
    iota16 = jnp.arange(16, dtype=jnp.int32)

    for r in range(_ROWS_PER_W):
        row = wid * _ROWS_PER_W + r
        pltpu.sync_copy(int_hbm.at[row], row_v)
        pltpu.sync_copy(sel_hbm.at[row], sel_v)

        def super_body(s, acc):
            m = _tree_max16(lambda c: row_v[pl.ds(s * 256 + c * 16, 16)])
            msup_v[pl.ds(s * 16, 16)] = m
            return jnp.maximum(acc, m)

        acc = lax.fori_loop(0, _NSUP, super_body,
                            jnp.full((16,), _NEG, jnp.float32))

        pos_list = []
        for k in range(5):
            gmax = jnp.max(acc)
            gv = jnp.full((16,), gmax)

            def fs_body(s, fs):
                hit = jnp.any(msup_v[pl.ds(s * 16, 16)] == gv)
                return jnp.where((fs == 999) & hit, s, fs)

            fs = lax.fori_loop(0, _NSUP, fs_body, jnp.int32(999))

            def fc_body(c, carry):
                fc, fl = carry
                eqm = row_v[pl.ds(fs * 256 + c * 16, 16)] == gv
                hit = jnp.any(eqm)
                lane = jnp.min(jnp.where(eqm, iota16, 99))
                take = (fc == 999) & hit
                return (jnp.where(take, c, fc), jnp.where(take, lane, fl))

            fc, fl = lax.fori_loop(0, _CPS, fc_body,
                                   (jnp.int32(999), jnp.int32(0)))

            base = fs * 256 + fc * 16
            pos_list.append(base + fl)

            v = row_v[pl.ds(base, 16)]
            row_v[pl.ds(base, 16)] = jnp.where(iota16 == fl, _NEG, v)
            msup_v[pl.ds(fs * 16, 16)] = _tree_max16(
                lambda c: row_v[pl.ds(fs * 256 + c * 16, 16)])
            acc = lax.fori_loop(
                0, _NSUP,
                lambda s, a: jnp.maximum(a, msup_v[pl.ds(s * 16, 16)]),
                jnp.full((16,), _NEG, jnp.float32))

        sv = sel_v[...]
        res = jnp.zeros((16,), jnp.int32)
        for rank in range(5):
            res = jnp.where(sv == rank, pos_list[rank], res)
        res = jnp.where(iota16 < 2, res, 0)
        res_v[...] = res
        pltpu.sync_copy(res_v, pos_hbm.at[row])


_sc_top5 = functools.partial(
    pl.kernel,
    mesh=plsc.VectorSubcoreMesh(core_axis_name="c", subcore_axis_name="s"),
    compiler_params=pltpu.CompilerParams(needs_layout_passes=False),
    out_type=jax.ShapeDtypeStruct((_B, 16), jnp.int32),
    scratch_types=[
        pltpu.VMEM((_S,), jnp.float32),
        pltpu.VMEM((_NSUP * 16,), jnp.float32),
        pltpu.VMEM((16,), jnp.int32),
        pltpu.VMEM((16,), jnp.int32),
    ],
)(_sc_body)

_ROWBLK = 8


def _tc_body(pos_ref, x_ref, out_ref, tok_ref):
    x = x_ref[...]
    pos = pos_ref[...]
    p0 = pos[:, 0:1]
    p1 = pos[:, 1:2]
    col = lax.broadcasted_iota(jnp.int32, (_ROWBLK, _S), 1)
    m0 = col == p0
    m1 = col == p1
    out_ref[...] = jnp.where(m0 | m1, jnp.float32(1.0), x)
    t0 = jnp.sum(jnp.where(m0, x, 0.0), axis=1, keepdims=True).astype(jnp.int32)
    t1 = jnp.sum(jnp.where(m1, x, 0.0), axis=1, keepdims=True).astype(jnp.int32)
    col8 = lax.broadcasted_iota(jnp.int32, (_ROWBLK, 8), 1)
    tok_ref[...] = jnp.where(col8 == 0, t0, jnp.where(col8 == 1, t1, 0))


_tc_mask = pl.pallas_call(
    _tc_body,
    grid=(_B // _ROWBLK,),
    in_specs=[
        pl.BlockSpec((_ROWBLK, 16), lambda i: (i, 0)),
        pl.BlockSpec((_ROWBLK, _S), lambda i: (i, 0)),
    ],
    out_specs=[
        pl.BlockSpec((_ROWBLK, _S), lambda i: (i, 0)),
        pl.BlockSpec((_ROWBLK, 8), lambda i: (i, 0)),
    ],
    out_shape=[
        jax.ShapeDtypeStruct((_B, _S), jnp.float32),
        jax.ShapeDtypeStruct((_B, 8), jnp.int32),
    ],
)


def kernel(x, intensity_):
    intens = jnp.squeeze(intensity_, axis=1)
    sel16 = _sel16_traced()
    pos16 = _sc_top5(intens, sel16)
    mask_x, tok = _tc_mask(pos16, x)
    return (mask_x, tok, pos16[:, :8])


# speedup vs baseline: 6.8674x; 1.0269x over previous
import functools

import jax
import jax.numpy as jnp
import numpy as np
from jax import lax
from jax.experimental import pallas as pl
from jax.experimental.pallas import tpu as pltpu
from jax.experimental.pallas import tpu_sc as plsc

_B = 128
_S = 8192
_NCHUNK = _S // 16
_NSUP = 32
_CPS = _NCHUNK // _NSUP
_ROWS_PER_W = 4
_NEG = float("-inf")


_SEL_PACKED = (
    "7A71>4CG9C9@E;:>18>?>51G==:A4@1A5ECC79>>;15DD35C27??A2FD?5@41C2>=95G@DC"
    "727G@C779A@4>FD5=4D5DD@54773:31G:@@759CGG81@=8A@@792:21EG"
)


def _sel16_const() -> np.ndarray:
    v = np.frombuffer(_SEL_PACKED.encode(), np.uint8).astype(np.int32) - 48
    out = np.zeros((_B, 16), np.int32)
    out[:, 0] = v // 5
    out[:, 1] = v % 5
    return out


def _tree_max16(load):
    vs = [load(c) for c in range(16)]
    while len(vs) > 1:
        vs = [jnp.maximum(vs[i], vs[i + 1]) for i in range(0, len(vs), 2)]
    return vs[0]


def _sc_body(int_hbm, sel_hbm, pos_hbm, row_v, msup_v, sel_v, res_v):
    wid = lax.axis_index("s") * 2 + lax.axis_index("c")
    iota16 = jnp.arange(16, dtype=jnp.int32)

    for r in range(_ROWS_PER_W):
        row = wid * _ROWS_PER_W + r
        pltpu.sync_copy(int_hbm.at[row], row_v)
        pltpu.sync_copy(sel_hbm.at[row], sel_v)

        def super_body(s, acc):
            m = _tree_max16(lambda c: row_v[pl.ds(s * 256 + c * 16, 16)])
            msup_v[pl.ds(s * 16, 16)] = m
            return jnp.maximum(acc, m)

        acc = lax.fori_loop(0, _NSUP, super_body,
                            jnp.full((16,), _NEG, jnp.float32))

        pos_list = []
        for k in range(5):
            gmax = jnp.max(acc)
            gv = jnp.full((16,), gmax)

            def fs_body(s, fs):
                hit = jnp.any(msup_v[pl.ds(s * 16, 16)] == gv)
                return jnp.where((fs == 999) & hit, s, fs)

            fs = lax.fori_loop(0, _NSUP, fs_body, jnp.int32(999))

            def fc_body(c, carry):
                fc, fl = carry
                eqm = row_v[pl.ds(fs * 256 + c * 16, 16)] == gv
                hit = jnp.any(eqm)
                lane = jnp.min(jnp.where(eqm, iota16, 99))
                take = (fc == 999) & hit
                return (jnp.where(take, c, fc), jnp.where(take, lane, fl))

            fc, fl = lax.fori_loop(0, _CPS, fc_body,
                                   (jnp.int32(999), jnp.int32(0)))

            base = fs * 256 + fc * 16
            pos_list.append(base + fl)

            v = row_v[pl.ds(base, 16)]
            row_v[pl.ds(base, 16)] = jnp.where(iota16 == fl, _NEG, v)
            msup_v[pl.ds(fs * 16, 16)] = _tree_max16(
                lambda c: row_v[pl.ds(fs * 256 + c * 16, 16)])
            acc = lax.fori_loop(
                0, _NSUP,
                lambda s, a: jnp.maximum(a, msup_v[pl.ds(s * 16, 16)]),
                jnp.full((16,), _NEG, jnp.float32))

        sv = sel_v[...]
        res = jnp.zeros((16,), jnp.int32)
        for rank in range(5):
            res = jnp.where(sv == rank, pos_list[rank], res)
        res = jnp.where(iota16 < 2, res, 0)
        res_v[...] = res
        pltpu.sync_copy(res_v, pos_hbm.at[row])


_sc_top5 = functools.partial(
    pl.kernel,
    mesh=plsc.VectorSubcoreMesh(core_axis_name="c", subcore_axis_name="s"),
    compiler_params=pltpu.CompilerParams(needs_layout_passes=False),
    out_type=jax.ShapeDtypeStruct((_B, 16), jnp.int32),
    scratch_types=[
        pltpu.VMEM((_S,), jnp.float32),
        pltpu.VMEM((_NSUP * 16,), jnp.float32),
        pltpu.VMEM((16,), jnp.int32),
        pltpu.VMEM((16,), jnp.int32),
    ],
)(_sc_body)

_ROWBLK = 8


def _tc_body(pos_ref, x_ref, out_ref, tok_ref):
    x = x_ref[...]
    pos = pos_ref[...]
    p0 = pos[:, 0:1]
    p1 = pos[:, 1:2]
    col = lax.broadcasted_iota(jnp.int32, (_ROWBLK, _S), 1)
    m0 = col == p0
    m1 = col == p1
    out_ref[...] = jnp.where(m0 | m1, jnp.float32(1.0), x)
    t0 = jnp.sum(jnp.where(m0, x, 0.0), axis=1, keepdims=True).astype(jnp.int32)
    t1 = jnp.sum(jnp.where(m1, x, 0.0), axis=1, keepdims=True).astype(jnp.int32)
    col8 = lax.broadcasted_iota(jnp.int32, (_ROWBLK, 8), 1)
    tok_ref[...] = jnp.where(col8 == 0, t0, jnp.where(col8 == 1, t1, 0))


_tc_mask = pl.pallas_call(
    _tc_body,
    grid=(_B // _ROWBLK,),
    in_specs=[
        pl.BlockSpec((_ROWBLK, 16), lambda i: (i, 0)),
        pl.BlockSpec((_ROWBLK, _S), lambda i: (i, 0)),
    ],
    out_specs=[
        pl.BlockSpec((_ROWBLK, _S), lambda i: (i, 0)),
        pl.BlockSpec((_ROWBLK, 8), lambda i: (i, 0)),
    ],
    out_shape=[
        jax.ShapeDtypeStruct((_B, _S), jnp.float32),
        jax.ShapeDtypeStruct((_B, 8), jnp.int32),
    ],
)


def kernel(x, intensity_):
    intens = jnp.squeeze(intensity_, axis=1)
    sel16 = jnp.asarray(_sel16_const())
    pos16 = _sc_top5(intens, sel16)
    mask_x, tok = _tc_mask(pos16, x)
    return (mask_x, tok, pos16[:, :8])


# speedup vs baseline: 11.5922x; 1.6880x over previous
import functools

import jax
import jax.numpy as jnp
import numpy as np
from jax import lax
from jax.experimental import pallas as pl
from jax.experimental.pallas import tpu as pltpu
from jax.experimental.pallas import tpu_sc as plsc

_B = 128
_S = 8192
_NSUP = 32
_ROWS_PER_W = 4
_NEG = float("-inf")

_SEL_PACKED = (
    "7A71>4CG9C9@E;:>18>?>51G==:A4@1A5ECC79>>;15DD35C27??A2FD?5@41C2>=95G@DC"
    "727G@C779A@4>FD5=4D5DD@54773:31G:@@759CGG81@=8A@@792:21EG"
)


def _sel16_const() -> np.ndarray:
    v = np.frombuffer(_SEL_PACKED.encode(), np.uint8).astype(np.int32) - 48
    out = np.zeros((_B, 16), np.int32)
    out[:, 0] = v // 5
    out[:, 1] = v % 5
    return out


def _tree_max16(load):
    vs = [load(c) for c in range(16)]
    while len(vs) > 1:
        vs = [jnp.maximum(vs[i], vs[i + 1]) for i in range(0, len(vs), 2)]
    return vs[0]


def _sc_body(int_hbm, x_hbm, sel_hbm, mx_hbm, tok_hbm, pos_hbm,
             ibuf, xbuf, sel4_v, tok4_v, res4_v,
             sem_i, sem_x, sem_o):
    wid = lax.axis_index("s") * 2 + lax.axis_index("c")
    iota16 = jnp.arange(16, dtype=jnp.int32)
    row0 = wid * _ROWS_PER_W

    for r in range(_ROWS_PER_W):
        pltpu.async_copy(int_hbm.at[row0 + r, 0],
                         ibuf.at[pl.ds(r * _S, _S)], sem_i.at[r])
        pltpu.async_copy(x_hbm.at[row0 + r],
                         xbuf.at[pl.ds(r * _S, _S)], sem_x.at[r])
    pltpu.sync_copy(sel_hbm.at[pl.ds(row0 * 16, _ROWS_PER_W * 16)], sel4_v)

    for r in range(_ROWS_PER_W):
        row = row0 + r
        rbase = r * _S
        pltpu.make_async_copy(int_hbm.at[row, 0],
                              ibuf.at[pl.ds(rbase, _S)], sem_i.at[r]).wait()

        def super_body(s, carry):
            s0a, s1a = carry
            mx = jnp.max(_tree_max16(
                lambda c: ibuf[pl.ds(rbase + s * 256 + c * 16, 16)]))
            mv = jnp.full((16,), mx)
            s0a = jnp.where(iota16 == s, mv, s0a)
            s1a = jnp.where(iota16 == s - 16, mv, s1a)
            return (s0a, s1a)

        s0, s1 = lax.fori_loop(
            0, _NSUP, super_body,
            (jnp.full((16,), _NEG, jnp.float32),
             jnp.full((16,), _NEG, jnp.float32)))

        pos_list = []
        for k in range(5):
            gmax = jnp.max(jnp.maximum(s0, s1))
            gv = jnp.full((16,), gmax)
            f0 = plsc.all_reduce_ffs(s0 == gv)
            f1 = plsc.all_reduce_ffs(s1 == gv)
            fs = jnp.min(jnp.where(f0 < 16, f0, 16 + f1))

            def fc_body(c, carry):
                fc_v, fl_v = carry
                eqm = ibuf[pl.ds(rbase + fs * 256 + c * 16, 16)] == gv
                l = plsc.all_reduce_ffs(eqm)
                take = (fc_v == 999) & (l < 16)
                return (jnp.where(take, c, fc_v), jnp.where(take, l, fl_v))

            fc_v, fl_v = lax.fori_loop(
                0, 16, fc_body,
                (jnp.full((16,), 999, jnp.int32),
                 jnp.full((16,), 0, jnp.int32)))
            fc = jnp.min(fc_v)
            fl = jnp.min(fl_v)

            base = fs * 256 + fc * 16
            pos_list.append(base + fl)

            v = ibuf[pl.ds(rbase + base, 16)]
            ibuf[pl.ds(rbase + base, 16)] = jnp.where(iota16 == fl, _NEG, v)
            if k < 4:
                mx = jnp.max(_tree_max16(
                    lambda c: ibuf[pl.ds(rbase + fs * 256 + c * 16, 16)]))
                mv = jnp.full((16,), mx)
                s0 = jnp.where(iota16 == fs, mv, s0)
                s1 = jnp.where(iota16 == fs - 16, mv, s1)

        sv = sel4_v[pl.ds(r * 16, 16)]
        mp = jnp.zeros((16,), jnp.int32)
        for rank in range(5):
            mp = jnp.where(sv == rank, pos_list[rank], mp)
        lane01 = iota16 < 2
        mp = jnp.where(lane01, mp, 0)
        res4_v[pl.ds(r * 16, 16)] = mp

        pltpu.make_async_copy(x_hbm.at[row],
                              xbuf.at[pl.ds(rbase, _S)], sem_x.at[r]).wait()
        gathered = plsc.load_gather(xbuf, [rbase + mp], mask=lane01)
        tok4_v[pl.ds(r * 16, 16)] = jnp.where(
            lane01, gathered.astype(jnp.int32), 0)
        plsc.store_scatter(xbuf, [rbase + mp],
                           jnp.full((16,), 1.0, jnp.float32), mask=lane01)
        pltpu.async_copy(xbuf.at[pl.ds(rbase, _S)], mx_hbm.at[row],
                         sem_o.at[r])

    pltpu.sync_copy(res4_v, pos_hbm.at[pl.ds(row0 * 16, _ROWS_PER_W * 16)])
    pltpu.sync_copy(tok4_v, tok_hbm.at[pl.ds(row0 * 16, _ROWS_PER_W * 16)])
    for r in range(_ROWS_PER_W):
        pltpu.make_async_copy(xbuf.at[pl.ds(r * _S, _S)],
                              mx_hbm.at[row0 + r], sem_o.at[r]).wait()


_sc_fused = functools.partial(
    pl.kernel,
    mesh=plsc.VectorSubcoreMesh(core_axis_name="c", subcore_axis_name="s"),
    compiler_params=pltpu.CompilerParams(needs_layout_passes=False),
    out_type=[
        jax.ShapeDtypeStruct((_B, _S), jnp.float32),
        jax.ShapeDtypeStruct((_B * 16,), jnp.int32),
        jax.ShapeDtypeStruct((_B * 16,), jnp.int32),
    ],
    scratch_types=[
        pltpu.VMEM((_ROWS_PER_W * _S,), jnp.float32),
        pltpu.VMEM((_ROWS_PER_W * _S,), jnp.float32),
        pltpu.VMEM((_ROWS_PER_W * 16,), jnp.int32),
        pltpu.VMEM((_ROWS_PER_W * 16,), jnp.int32),
        pltpu.VMEM((_ROWS_PER_W * 16,), jnp.int32),
        pltpu.SemaphoreType.DMA((_ROWS_PER_W,)),
        pltpu.SemaphoreType.DMA((_ROWS_PER_W,)),
        pltpu.SemaphoreType.DMA((_ROWS_PER_W,)),
    ],
)(_sc_body)


def kernel(x, intensity_):
    sel16 = jnp.asarray(_sel16_const()).reshape(_B * 16)
    mask_x, tok_flat, pos_flat = _sc_fused(intensity_, x, sel16)
    tok16 = tok_flat.reshape(_B, 16)
    pos16 = pos_flat.reshape(_B, 16)
    return (mask_x, tok16[:, :8], pos16[:, :8])
